# baseline (device time: 15046 ns/iter reference)
import jax
import jax.numpy as jnp
from jax import lax
from jax.experimental import pallas as pl
from jax.experimental.pallas import tpu as pltpu

N_DEV = 32


def kernel(x, dy, gamma):
    m_per, d = x.shape

    offsets = sorted(range(1, N_DEV), key=lambda o: -min(o, N_DEV - o))

    def body(x_hbm, dy_hbm, out_ref,
             xv_ref, dyv_ref, comm_ref, cp_sems, send_sems, recv_sems):
        my = lax.axis_index("i")

        barrier_sem = pltpu.get_barrier_semaphore()
        for off in offsets:
            peer = lax.rem(my + off, N_DEV)
            pl.semaphore_signal(
                barrier_sem, inc=1,
                device_id=(peer,), device_id_type=pl.DeviceIdType.MESH,
            )

        cp_x = pltpu.make_async_copy(x_hbm, xv_ref, cp_sems.at[0])
        cp_dy = pltpu.make_async_copy(dy_hbm, dyv_ref, cp_sems.at[1])
        cp_x.start()
        cp_dy.start()
        cp_x.wait()
        cp_dy.wait()

        xv = xv_ref[:, :]
        dyv = dyv_ref[:, :]
        inv_d = 1.0 / d
        s1 = jnp.sum(xv, axis=1, keepdims=True) * inv_d
        s2 = jnp.sum(xv * xv, axis=1, keepdims=True) * inv_d
        a = lax.rsqrt(s2 - s1 * s1 + 1e-5)
        c = a * s1
        pg = jnp.sum(dyv * (a * xv - c), axis=0, keepdims=True)
        pb = jnp.sum(dyv, axis=0, keepdims=True)
        comm_ref[pl.ds(my, 1), :] = jnp.concatenate([pg, pb], axis=1)

        pl.semaphore_wait(barrier_sem, N_DEV - 1)

        rdmas = []
        for off in offsets:
            peer = lax.rem(my + off, N_DEV)
            rdma = pltpu.make_async_remote_copy(
                src_ref=comm_ref.at[pl.ds(my, 1)],
                dst_ref=comm_ref.at[pl.ds(my, 1)],
                send_sem=send_sems.at[off - 1],
                recv_sem=recv_sems.at[off - 1],
                device_id=(peer,),
                device_id_type=pl.DeviceIdType.MESH,
            )
            rdma.start()
            rdmas.append(rdma)

        for rdma in rdmas:
            rdma.wait_recv()

        total = jnp.sum(comm_ref[:, :], axis=0, keepdims=True)
        out_ref[pl.ds(0, 1), :] = total[:, :d]
        out_ref[pl.ds(1, 1), :] = total[:, d:]

        for rdma in rdmas:
            rdma.wait_send()

    return pl.pallas_call(
        body,
        out_shape=jax.ShapeDtypeStruct((2, d), jnp.float32),
        in_specs=[
            pl.BlockSpec(memory_space=pl.ANY),
            pl.BlockSpec(memory_space=pl.ANY),
        ],
        out_specs=pl.BlockSpec(memory_space=pltpu.VMEM),
        scratch_shapes=[
            pltpu.VMEM((m_per, d), jnp.float32),
            pltpu.VMEM((m_per, d), jnp.float32),
            pltpu.VMEM((N_DEV, 2 * d), jnp.float32),
            pltpu.SemaphoreType.DMA((2,)),
            pltpu.SemaphoreType.DMA((N_DEV - 1,)),
            pltpu.SemaphoreType.DMA((N_DEV - 1,)),
        ],
        compiler_params=pltpu.CompilerParams(collective_id=0),
    )(x, dy)
